# Initial kernel scaffold; baseline (speedup 1.0000x reference)
import jax
import jax.numpy as jnp
from jax.experimental import pallas as pl


def kernel(input_ids, word_emb, pos_emb, gamma, beta):
    raise NotImplementedError("write your pallas kernel here")


# speedup vs baseline: 2.7129x; 2.7129x over previous
import functools

import jax
import jax.numpy as jnp
from jax import lax
from jax.experimental import pallas as pl
from jax.experimental.pallas import tpu as pltpu
from jax.experimental.pallas import tpu_sc as plsc

VOCAB = 100000
DIM = 128
PAD = 1
B = 1024
L = 200
EPS = 1e-5

NC = 2
NS = 16
NW = NC * NS
ROWS_PER_W = B // NW
LP = 208
NCHUNK = LP // 16
NPOS = L + 2
KD = DIM // 16

_MAGIC = 0x5F3759DF
_INV_D = 1.0 / DIM


def _extract(vec, onehot, zero):
    return jnp.sum(jnp.where(onehot, vec, zero))


def _sc_body(ids_hbm, word_hbm, posf_hbm, gamma_hbm, beta_hbm, out_hbm,
             ids_v, idx_a, idx_b, posid_v, maskf_v, postab_v, wbuf_v, obuf_v,
             gv, bv, sem_a, sem_b):
    wid = lax.axis_index("s") * NC + lax.axis_index("c")
    lane = lax.iota(jnp.int32, 16)

    pltpu.sync_copy(posf_hbm.at[pl.ds(0, NPOS * DIM)], postab_v)
    pltpu.sync_copy(gamma_hbm, gv)
    pltpu.sync_copy(beta_hbm, bv)
    g = [gv[pl.ds(k * 16, 16)] for k in range(KD)]
    b = [bv[pl.ds(k * 16, 16)] for k in range(KD)]

    def row_body(r, carry0):
        rb = wid * ROWS_PER_W + r
        pltpu.sync_copy(ids_hbm.at[pl.ds(rb * L, L)], ids_v.at[pl.ds(0, L)])
        tail = ids_v[pl.ds(192, 16)]
        ids_v[pl.ds(192, 16)] = jnp.where(lane < 8, tail, PAD)

        carry = jnp.int32(0)
        for c in range(NCHUNK):
            iv = ids_v[pl.ds(c * 16, 16)]
            if c < 8:
                idx_a[pl.ds(c * 16, 16)] = iv
            else:
                idx_b[pl.ds((c - 8) * 16, 16)] = iv
            m = (iv != PAD).astype(jnp.int32)
            s = jnp.cumsum(m)
            posid_v[pl.ds(c * 16, 16)] = (s + carry) * m + PAD
            maskf_v[pl.ds(c * 16, 16)] = m.astype(jnp.float32)
            carry = carry + jnp.sum(m)

        ga = pltpu.async_copy(word_hbm.at[idx_a], wbuf_v.at[pl.ds(0, 128)],
                              sem_a)
        gb = pltpu.async_copy(word_hbm.at[idx_b], wbuf_v.at[pl.ds(128, 80)],
                              sem_b)
        ga.wait()
        gb.wait()

        def token(c, j):
            t = c * 16 + j
            onehot = lane == j
            pidv = posid_v[pl.ds(c * 16, 16)]
            mv = maskf_v[pl.ds(c * 16, 16)]
            pid = _extract(pidv, onehot, jnp.int32(0))
            msk = jnp.broadcast_to(_extract(mv, onehot, jnp.float32(0.0)),
                                   (16,))
            base = pid * DIM
            e = []
            for k in range(KD):
                w = wbuf_v[t, pl.ds(k * 16, 16)]
                p = postab_v[pl.ds(base + k * 16, 16)]
                e.append(w * msk + p)
            s01 = (e[0] + e[1]) + (e[2] + e[3])
            s23 = (e[4] + e[5]) + (e[6] + e[7])
            tot = jnp.broadcast_to(jnp.sum(s01 + s23), (16,))
            q01 = (e[0] * e[0] + e[1] * e[1]) + (e[2] * e[2] + e[3] * e[3])
            q23 = (e[4] * e[4] + e[5] * e[5]) + (e[6] * e[6] + e[7] * e[7])
            totq = jnp.broadcast_to(jnp.sum(q01 + q23), (16,))
            mu = tot * _INV_D
            var = totq * _INV_D - mu * mu
            x = var + EPS
            i = lax.bitcast_convert_type(x, jnp.int32)
            y = lax.bitcast_convert_type(
                jnp.int32(_MAGIC) - lax.shift_right_arithmetic(i, 1),
                jnp.float32)
            y = y * (1.5 - 0.5 * x * y * y)
            y = y * (1.5 - 0.5 * x * y * y)
            y = y * (1.5 - 0.5 * x * y * y)
            c2 = -mu * y
            for k in range(KD):
                o = e[k] * y + c2
                obuf_v[pl.ds(t * DIM + k * 16, 16)] = o * g[k] + b[k]

        def chunk_body(c, carry1):
            for j in range(16):
                token(c, j)
            return carry1

        lax.fori_loop(0, NCHUNK - 1, chunk_body, 0)
        for j in range(16):
            token(NCHUNK - 1, j)

        pltpu.sync_copy(obuf_v.at[pl.ds(0, L * DIM)],
                        out_hbm.at[pl.ds(rb * L * DIM, L * DIM)])
        return carry0

    lax.fori_loop(0, ROWS_PER_W, row_body, 0)


@jax.jit
def _run(input_ids, word_emb, pos_flat, gamma, beta):
    mesh = plsc.VectorSubcoreMesh(core_axis_name="c", subcore_axis_name="s")
    f = pl.kernel(
        _sc_body,
        out_type=jax.ShapeDtypeStruct((B * L * DIM,), jnp.float32),
        mesh=mesh,
        scratch_types=[
            pltpu.VMEM((LP,), jnp.int32),
            pltpu.VMEM((128,), jnp.int32),
            pltpu.VMEM((80,), jnp.int32),
            pltpu.VMEM((LP,), jnp.int32),
            pltpu.VMEM((LP,), jnp.float32),
            pltpu.VMEM((NPOS * DIM,), jnp.float32),
            pltpu.VMEM((LP, DIM), jnp.float32),
            pltpu.VMEM((LP * DIM,), jnp.float32),
            pltpu.VMEM((DIM,), jnp.float32),
            pltpu.VMEM((DIM,), jnp.float32),
            pltpu.SemaphoreType.DMA,
            pltpu.SemaphoreType.DMA,
        ],
        compiler_params=pltpu.CompilerParams(needs_layout_passes=False),
    )
    return f(input_ids, word_emb, pos_flat, gamma, beta)


def kernel(input_ids, word_emb, pos_emb, gamma, beta):
    out = _run(input_ids.astype(jnp.int32).reshape(-1), word_emb,
               pos_emb.reshape(-1), gamma, beta)
    return out.reshape(B, L, DIM)


# speedup vs baseline: 2.8333x; 1.0444x over previous
import jax
import jax.numpy as jnp
from jax import lax
from jax.experimental import pallas as pl
from jax.experimental.pallas import tpu as pltpu
from jax.experimental.pallas import tpu_sc as plsc

VOCAB = 100000
DIM = 128
PAD = 1
B = 1024
L = 200
EPS = 1e-5

NC = 2
NS = 16
NW = NC * NS
ROWS_PER_W = B // NW
LP = 208
NCHUNK = LP // 16
NPOS = L + 2
KD = DIM // 16

_MAGIC = 0x5F3759DF
_INV_D = 1.0 / DIM


def _extract(vec, onehot, zero):
    return jnp.sum(jnp.where(onehot, vec, zero))


def _sc_body(ids_hbm, word_hbm, posf_hbm, gamma_hbm, beta_hbm, out_hbm,
             ids_v, idx_a0, idx_b0, idx_a1, idx_b1,
             posid0, maskf0, posid1, maskf1, postab_v, wbuf0, wbuf1,
             gv, bv, sem_ga0, sem_gb0, sem_ga1, sem_gb1, sem_o0, sem_o1):
    wid = lax.axis_index("s") * NC + lax.axis_index("c")
    lane = lax.iota(jnp.int32, 16)

    pltpu.sync_copy(posf_hbm.at[pl.ds(0, NPOS * DIM)], postab_v)
    pltpu.sync_copy(gamma_hbm, gv)
    pltpu.sync_copy(beta_hbm, bv)
    g = [gv[pl.ds(k * 16, 16)] for k in range(KD)]
    b = [bv[pl.ds(k * 16, 16)] for k in range(KD)]

    bufs = (
        (idx_a0, idx_b0, posid0, maskf0, wbuf0, sem_ga0, sem_gb0, sem_o0),
        (idx_a1, idx_b1, posid1, maskf1, wbuf1, sem_ga1, sem_gb1, sem_o1),
    )

    def prep(r, bi):
        idx_a, idx_b, posid_v, maskf_v, wbuf_v, sem_ga, sem_gb, _ = bufs[bi]
        rb = wid * ROWS_PER_W + r
        pltpu.sync_copy(ids_hbm.at[pl.ds(rb * L, L)], ids_v.at[pl.ds(0, L)])
        tail = ids_v[pl.ds(192, 16)]
        ids_v[pl.ds(192, 16)] = jnp.where(lane < 8, tail, PAD)

        carry = jnp.int32(0)
        for c in range(NCHUNK):
            iv = ids_v[pl.ds(c * 16, 16)]
            if c < 8:
                idx_a[pl.ds(c * 16, 16)] = iv
            else:
                idx_b[pl.ds((c - 8) * 16, 16)] = iv
            m = (iv != PAD).astype(jnp.int32)
            s = jnp.cumsum(m)
            posid_v[pl.ds(c * 16, 16)] = (s + carry) * m + PAD
            maskf_v[pl.ds(c * 16, 16)] = m.astype(jnp.float32)
            carry = carry + jnp.sum(m)

        pltpu.async_copy(word_hbm.at[idx_a], wbuf_v.at[pl.ds(0, 128)],
                         sem_ga)
        pltpu.async_copy(word_hbm.at[idx_b], wbuf_v.at[pl.ds(128, 80)],
                         sem_gb)

    def wait_gather(bi):
        idx_a, idx_b, _, _, wbuf_v, sem_ga, sem_gb, _ = bufs[bi]
        pltpu.make_async_copy(word_hbm.at[idx_a], wbuf_v.at[pl.ds(0, 128)],
                              sem_ga).wait()
        pltpu.make_async_copy(word_hbm.at[idx_b], wbuf_v.at[pl.ds(128, 80)],
                              sem_gb).wait()

    def fire_out(r, bi):
        _, _, _, _, wbuf_v, _, _, sem_o = bufs[bi]
        rb = wid * ROWS_PER_W + r
        pltpu.async_copy(wbuf_v.at[pl.ds(0, L)],
                         out_hbm.at[pl.ds(rb * L, L)], sem_o)

    def wait_out(bi):
        _, _, _, _, wbuf_v, _, _, sem_o = bufs[bi]
        pltpu.make_async_copy(wbuf_v.at[pl.ds(0, L)],
                              out_hbm.at[pl.ds(0, L)], sem_o).wait()

    def token(c, j, bi):
        _, _, posid_v, maskf_v, wbuf_v, _, _, _ = bufs[bi]
        t = c * 16 + j
        onehot = lane == j
        pidv = posid_v[pl.ds(c * 16, 16)]
        mv = maskf_v[pl.ds(c * 16, 16)]
        pid = _extract(pidv, onehot, jnp.int32(0))
        msk = jnp.broadcast_to(_extract(mv, onehot, jnp.float32(0.0)),
                               (16,))
        base = pid * DIM
        e = []
        for k in range(KD):
            w = wbuf_v[t, pl.ds(k * 16, 16)]
            p = postab_v[pl.ds(base + k * 16, 16)]
            e.append(w * msk + p)
        s01 = (e[0] + e[1]) + (e[2] + e[3])
        s23 = (e[4] + e[5]) + (e[6] + e[7])
        tot = jnp.broadcast_to(jnp.sum(s01 + s23), (16,))
        q01 = (e[0] * e[0] + e[1] * e[1]) + (e[2] * e[2] + e[3] * e[3])
        q23 = (e[4] * e[4] + e[5] * e[5]) + (e[6] * e[6] + e[7] * e[7])
        totq = jnp.broadcast_to(jnp.sum(q01 + q23), (16,))
        mu = tot * _INV_D
        var = totq * _INV_D - mu * mu
        x = var + EPS
        i = lax.bitcast_convert_type(x, jnp.int32)
        y = lax.bitcast_convert_type(
            jnp.int32(_MAGIC) - lax.shift_right_arithmetic(i, 1),
            jnp.float32)
        y = y * (1.5 - 0.5 * x * y * y)
        y = y * (1.5 - 0.5 * x * y * y)
        y = y * (1.5 - 0.5 * x * y * y)
        c2 = -mu * y
        for k in range(KD):
            o = e[k] * y + c2
            wbuf_v[t, pl.ds(k * 16, 16)] = o * g[k] + b[k]

    def compute(bi):
        def chunk_body(c, carry1):
            for j in range(16):
                token(c, j, bi)
            return carry1

        lax.fori_loop(0, NCHUNK - 1, chunk_body, 0)
        for j in range(16):
            token(NCHUNK - 1, j, bi)

    prep(0, 0)

    def pair_body(i, carry0):
        r0 = 2 * i

        @pl.when(i > 0)
        def _():
            wait_out(1)

        prep(r0 + 1, 1)
        wait_gather(0)
        compute(0)
        fire_out(r0, 0)

        @pl.when(i < ROWS_PER_W // 2 - 1)
        def _():
            wait_out(0)
            prep(r0 + 2, 0)

        wait_gather(1)
        compute(1)
        fire_out(r0 + 1, 1)
        return carry0

    lax.fori_loop(0, ROWS_PER_W // 2, pair_body, 0)
    wait_out(0)
    wait_out(1)


@jax.jit
def _run(input_ids, word_emb, pos_flat, gamma, beta):
    mesh = plsc.VectorSubcoreMesh(core_axis_name="c", subcore_axis_name="s")
    f = pl.kernel(
        _sc_body,
        out_type=jax.ShapeDtypeStruct((B * L, DIM), jnp.float32),
        mesh=mesh,
        scratch_types=[
            pltpu.VMEM((LP,), jnp.int32),
            pltpu.VMEM((128,), jnp.int32),
            pltpu.VMEM((80,), jnp.int32),
            pltpu.VMEM((128,), jnp.int32),
            pltpu.VMEM((80,), jnp.int32),
            pltpu.VMEM((LP,), jnp.int32),
            pltpu.VMEM((LP,), jnp.float32),
            pltpu.VMEM((LP,), jnp.int32),
            pltpu.VMEM((LP,), jnp.float32),
            pltpu.VMEM((NPOS * DIM,), jnp.float32),
            pltpu.VMEM((LP, DIM), jnp.float32),
            pltpu.VMEM((LP, DIM), jnp.float32),
            pltpu.VMEM((DIM,), jnp.float32),
            pltpu.VMEM((DIM,), jnp.float32),
            pltpu.SemaphoreType.DMA,
            pltpu.SemaphoreType.DMA,
            pltpu.SemaphoreType.DMA,
            pltpu.SemaphoreType.DMA,
            pltpu.SemaphoreType.DMA,
            pltpu.SemaphoreType.DMA,
        ],
        compiler_params=pltpu.CompilerParams(needs_layout_passes=False),
    )
    return f(input_ids, word_emb, pos_flat, gamma, beta)


def kernel(input_ids, word_emb, pos_emb, gamma, beta):
    out = _run(input_ids.astype(jnp.int32).reshape(-1), word_emb,
               pos_emb.reshape(-1), gamma, beta)
    return out.reshape(B, L, DIM)
